# initial kernel scaffold (unmeasured)
import jax
import jax.numpy as jnp
from jax import lax
from jax.experimental import pallas as pl
from jax.experimental.pallas import tpu as pltpu

N_DEV = 4
E_LOCAL = 8


def kernel(x, router_W, route_idx, expert_W):
    m, d = x.shape
    n_exp = router_W.shape[1]
    d_out = expert_W.shape[2]

    scores = jnp.dot(x, router_W)
    probs = jax.nn.softmax(scores, axis=-1)
    top = jnp.take_along_axis(probs, route_idx, axis=1)
    gnorm = top / jnp.sum(top, axis=1, keepdims=True)
    onehot = route_idx[:, :, None] == jnp.arange(n_exp)[None, None, :]
    gw = jnp.sum(jnp.where(onehot, gnorm[:, :, None], 0.0), axis=1)
    gw = gw.astype(jnp.float32)

    def body(
        x_ref, gw_ref, ew_ref, out_ref,
        xbuf, gwbuf, accbuf,
        x_send, x_recv, g_send, g_recv, a_send, a_recv,
        f_send, f_recv,
    ):
        my = lax.axis_index("i")
        right = lax.rem(my + 1, N_DEV)
        left = lax.rem(my + N_DEV - 1, N_DEV)
        base = my * E_LOCAL

        barrier = pltpu.get_barrier_semaphore()
        for nbr in (left, right):
            pl.semaphore_signal(
                barrier, inc=1,
                device_id=(nbr,), device_id_type=pl.DeviceIdType.MESH,
            )
        pl.semaphore_wait(barrier, 2)

        col = lax.broadcasted_iota(jnp.int32, (m, n_exp), 1)

        def contrib(x_v, gw_v):
            res = None
            for e in range(E_LOCAL):
                w = jnp.sum(
                    jnp.where(col == base + e, gw_v, 0.0),
                    axis=1, keepdims=True,
                )
                y = jnp.dot(x_v, ew_ref[e], preferred_element_type=jnp.float32)
                t = w * y
                res = t if res is None else res + t
            return res

        accbuf[0, :, :] = contrib(x_ref[...], gw_ref[...])

        for h in range(1, N_DEV):
            rs = h % 2
            ss = (h + 1) % 2
            if h == 1:
                x_src, g_src = x_ref, gw_ref
            else:
                x_src, g_src = xbuf.at[ss], gwbuf.at[ss]
            rdx = pltpu.make_async_remote_copy(
                src_ref=x_src, dst_ref=xbuf.at[rs],
                send_sem=x_send.at[rs], recv_sem=x_recv.at[rs],
                device_id=(right,), device_id_type=pl.DeviceIdType.MESH,
            )
            rdg = pltpu.make_async_remote_copy(
                src_ref=g_src, dst_ref=gwbuf.at[rs],
                send_sem=g_send.at[rs], recv_sem=g_recv.at[rs],
                device_id=(right,), device_id_type=pl.DeviceIdType.MESH,
            )
            rda = pltpu.make_async_remote_copy(
                src_ref=accbuf.at[ss], dst_ref=accbuf.at[rs],
                send_sem=a_send.at[rs], recv_sem=a_recv.at[rs],
                device_id=(right,), device_id_type=pl.DeviceIdType.MESH,
            )
            rdx.start()
            rdg.start()
            rda.start()
            rdx.wait()
            rdg.wait()
            rda.wait()
            accbuf[rs, :, :] = accbuf[rs, :, :] + contrib(
                xbuf[rs, :, :], gwbuf[rs, :, :]
            )

        final = pltpu.make_async_remote_copy(
            src_ref=accbuf.at[1], dst_ref=out_ref,
            send_sem=f_send, recv_sem=f_recv,
            device_id=(right,), device_id_type=pl.DeviceIdType.MESH,
        )
        final.start()
        final.wait()

    return pl.pallas_call(
        body,
        out_shape=jax.ShapeDtypeStruct((m, d_out), jnp.float32),
        in_specs=[
            pl.BlockSpec(memory_space=pltpu.VMEM),
            pl.BlockSpec(memory_space=pltpu.VMEM),
            pl.BlockSpec(memory_space=pltpu.VMEM),
        ],
        out_specs=pl.BlockSpec(memory_space=pltpu.VMEM),
        scratch_shapes=[
            pltpu.VMEM((2, m, d), jnp.float32),
            pltpu.VMEM((2, m, n_exp), jnp.float32),
            pltpu.VMEM((2, m, d_out), jnp.float32),
            pltpu.SemaphoreType.DMA((2,)),
            pltpu.SemaphoreType.DMA((2,)),
            pltpu.SemaphoreType.DMA((2,)),
            pltpu.SemaphoreType.DMA((2,)),
            pltpu.SemaphoreType.DMA((2,)),
            pltpu.SemaphoreType.DMA((2,)),
            pltpu.SemaphoreType.DMA,
            pltpu.SemaphoreType.DMA,
        ],
        compiler_params=pltpu.CompilerParams(collective_id=0),
    )(x, gw, expert_W)


# baseline (device time: 862771 ns/iter reference)
import jax
import jax.numpy as jnp
from jax import lax
from jax.experimental import pallas as pl
from jax.experimental.pallas import tpu as pltpu

N_DEV = 4
E_LOCAL = 8


def kernel(x, router_W, route_idx, expert_W):
    m, d = x.shape
    n_exp = router_W.shape[1]
    d_out = expert_W.shape[2]

    scores = jnp.dot(x, router_W)
    probs = jax.nn.softmax(scores, axis=-1)
    top = jnp.take_along_axis(probs, route_idx, axis=1)
    gnorm = top / jnp.sum(top, axis=1, keepdims=True)
    onehot = route_idx[:, :, None] == jnp.arange(n_exp)[None, None, :]
    gw = jnp.sum(jnp.where(onehot, gnorm[:, :, None], 0.0), axis=1)
    gw = gw.astype(jnp.float32)

    def body(
        x_ref, gw_ref, ew_ref, out_ref,
        xbuf, gwbuf, accbuf, wbuf,
        load_sem, w_sem,
        x_send, x_recv, g_send, g_recv, a_send, a_recv,
        f_send, f_recv,
    ):
        my = lax.axis_index("i")
        right = lax.rem(my + 1, N_DEV)
        left = lax.rem(my + N_DEV - 1, N_DEV)
        base = my * E_LOCAL

        cp_x = pltpu.make_async_copy(x_ref, xbuf.at[0], load_sem.at[0])
        cp_g = pltpu.make_async_copy(gw_ref, gwbuf.at[0], load_sem.at[1])
        cp_x.start()
        cp_g.start()

        barrier = pltpu.get_barrier_semaphore()
        for nbr in (left, right):
            pl.semaphore_signal(
                barrier, inc=1,
                device_id=(nbr,), device_id_type=pl.DeviceIdType.MESH,
            )
        pl.semaphore_wait(barrier, 2)

        cp_x.wait()
        cp_g.wait()

        col = lax.broadcasted_iota(jnp.int32, (m, n_exp), 1)

        def add_contrib(x_slot, gw_slot, acc_slot, first):
            if first:
                accbuf[acc_slot] = jnp.zeros((m, d_out), jnp.float32)
            pltpu.make_async_copy(
                ew_ref.at[0], wbuf.at[0], w_sem.at[0]
            ).start()

            def step(e, carry):
                nxt = e + 1
                slot = lax.rem(e, 2)
                nslot = lax.rem(nxt, 2)

                @pl.when(nxt < E_LOCAL)
                def _():
                    pltpu.make_async_copy(
                        ew_ref.at[nxt], wbuf.at[nslot], w_sem.at[nslot]
                    ).start()

                pltpu.make_async_copy(
                    ew_ref.at[e], wbuf.at[slot], w_sem.at[slot]
                ).wait()
                w = jnp.sum(
                    jnp.where(col == base + e, gwbuf[gw_slot], 0.0),
                    axis=1, keepdims=True,
                )
                y = jnp.dot(
                    xbuf[x_slot], wbuf[slot],
                    preferred_element_type=jnp.float32,
                )
                accbuf[acc_slot] = accbuf[acc_slot] + w * y
                return carry

            lax.fori_loop(0, E_LOCAL, step, 0)

        add_contrib(0, 0, 0, first=True)

        for h in range(1, N_DEV):
            rs = h % 2
            ss = (h + 1) % 2
            rdx = pltpu.make_async_remote_copy(
                src_ref=xbuf.at[ss], dst_ref=xbuf.at[rs],
                send_sem=x_send.at[rs], recv_sem=x_recv.at[rs],
                device_id=(right,), device_id_type=pl.DeviceIdType.MESH,
            )
            rdg = pltpu.make_async_remote_copy(
                src_ref=gwbuf.at[ss], dst_ref=gwbuf.at[rs],
                send_sem=g_send.at[rs], recv_sem=g_recv.at[rs],
                device_id=(right,), device_id_type=pl.DeviceIdType.MESH,
            )
            rda = pltpu.make_async_remote_copy(
                src_ref=accbuf.at[ss], dst_ref=accbuf.at[rs],
                send_sem=a_send.at[rs], recv_sem=a_recv.at[rs],
                device_id=(right,), device_id_type=pl.DeviceIdType.MESH,
            )
            rdx.start()
            rdg.start()
            rda.start()
            rdx.wait()
            rdg.wait()
            rda.wait()
            add_contrib(rs, rs, rs, first=False)

        final = pltpu.make_async_remote_copy(
            src_ref=accbuf.at[1], dst_ref=out_ref,
            send_sem=f_send, recv_sem=f_recv,
            device_id=(right,), device_id_type=pl.DeviceIdType.MESH,
        )
        final.start()
        final.wait()

    return pl.pallas_call(
        body,
        out_shape=jax.ShapeDtypeStruct((m, d_out), jnp.float32),
        in_specs=[
            pl.BlockSpec(memory_space=pl.ANY),
            pl.BlockSpec(memory_space=pl.ANY),
            pl.BlockSpec(memory_space=pl.ANY),
        ],
        out_specs=pl.BlockSpec(memory_space=pltpu.VMEM),
        scratch_shapes=[
            pltpu.VMEM((2, m, d), jnp.float32),
            pltpu.VMEM((2, m, n_exp), jnp.float32),
            pltpu.VMEM((2, m, d_out), jnp.float32),
            pltpu.VMEM((2, d, d_out), jnp.float32),
            pltpu.SemaphoreType.DMA((2,)),
            pltpu.SemaphoreType.DMA((2,)),
            pltpu.SemaphoreType.DMA((2,)),
            pltpu.SemaphoreType.DMA((2,)),
            pltpu.SemaphoreType.DMA((2,)),
            pltpu.SemaphoreType.DMA((2,)),
            pltpu.SemaphoreType.DMA((2,)),
            pltpu.SemaphoreType.DMA((2,)),
            pltpu.SemaphoreType.DMA,
            pltpu.SemaphoreType.DMA,
        ],
        compiler_params=pltpu.CompilerParams(
            collective_id=0,
            vmem_limit_bytes=58 * 1024 * 1024,
        ),
    )(x, gw, expert_W)


# device time: 530971 ns/iter; 1.6249x vs baseline; 1.6249x over previous
import jax
import jax.numpy as jnp
from jax import lax
from jax.experimental import pallas as pl
from jax.experimental.pallas import tpu as pltpu

N_DEV = 4
E_LOCAL = 8
R, L = 0, 1


def kernel(x, router_W, route_idx, expert_W):
    m, d = x.shape
    n_exp = router_W.shape[1]
    d_out = expert_W.shape[2]
    half = m // 2

    scores = jnp.dot(x, router_W)
    probs = jax.nn.softmax(scores, axis=-1)
    top = jnp.take_along_axis(probs, route_idx, axis=1)
    gnorm = top / jnp.sum(top, axis=1, keepdims=True)
    onehot = route_idx[:, :, None] == jnp.arange(n_exp)[None, None, :]
    gw = jnp.sum(jnp.where(onehot, gnorm[:, :, None], 0.0), axis=1)
    gw = gw.astype(jnp.float32)

    def body(
        x_ref, gw_ref, ew_ref, out_ref,
        xbuf, gbuf, abuf, wbuf,
        load_sem, w_sem,
        xs, xr, gs, gr, as_, ar, fs, fr,
    ):
        my = lax.axis_index("i")
        right = lax.rem(my + 1, N_DEV)
        left = lax.rem(my + N_DEV - 1, N_DEV)
        nbr = (right, left)
        base = my * E_LOCAL

        loads = [
            pltpu.make_async_copy(
                x_ref.at[pl.ds(0, half)], xbuf.at[R, 0], load_sem.at[0]),
            pltpu.make_async_copy(
                x_ref.at[pl.ds(half, half)], xbuf.at[L, 0], load_sem.at[1]),
            pltpu.make_async_copy(
                gw_ref.at[pl.ds(0, half)], gbuf.at[R, 0], load_sem.at[2]),
            pltpu.make_async_copy(
                gw_ref.at[pl.ds(half, half)], gbuf.at[L, 0], load_sem.at[3]),
        ]
        for cp in loads:
            cp.start()

        barrier = pltpu.get_barrier_semaphore()
        for tgt in (left, right):
            pl.semaphore_signal(
                barrier, inc=1,
                device_id=(tgt,), device_id_type=pl.DeviceIdType.MESH,
            )
        pl.semaphore_wait(barrier, 2)

        for cp in loads:
            cp.wait()

        col = lax.broadcasted_iota(jnp.int32, (half, n_exp), 1)

        def add_round(slot, first):
            if first:
                abuf[R, slot] = jnp.zeros((half, d_out), jnp.float32)
                abuf[L, slot] = jnp.zeros((half, d_out), jnp.float32)
            pltpu.make_async_copy(
                ew_ref.at[0], wbuf.at[0], w_sem.at[0]
            ).start()

            def step(e, carry):
                nxt = e + 1
                wslot = lax.rem(e, 2)
                nwslot = lax.rem(nxt, 2)

                @pl.when(nxt < E_LOCAL)
                def _():
                    pltpu.make_async_copy(
                        ew_ref.at[nxt], wbuf.at[nwslot], w_sem.at[nwslot]
                    ).start()

                pltpu.make_async_copy(
                    ew_ref.at[e], wbuf.at[wslot], w_sem.at[wslot]
                ).wait()
                for dir_ in (R, L):
                    w = jnp.sum(
                        jnp.where(col == base + e, gbuf[dir_, slot], 0.0),
                        axis=1, keepdims=True,
                    )
                    y = jnp.dot(
                        xbuf[dir_, slot], wbuf[wslot],
                        preferred_element_type=jnp.float32,
                    )
                    abuf[dir_, slot] = abuf[dir_, slot] + w * y
                return carry

            lax.fori_loop(0, E_LOCAL, step, 0)

        add_round(0, first=True)

        for h in range(1, N_DEV):
            rs = h % 2
            ss = (h + 1) % 2
            rdmas = []
            for buf, send_sems, recv_sems in (
                (gbuf, gs, gr), (xbuf, xs, xr), (abuf, as_, ar),
            ):
                for dir_ in (R, L):
                    rdmas.append(pltpu.make_async_remote_copy(
                        src_ref=buf.at[dir_, ss], dst_ref=buf.at[dir_, rs],
                        send_sem=send_sems.at[dir_, rs],
                        recv_sem=recv_sems.at[dir_, rs],
                        device_id=(nbr[dir_],),
                        device_id_type=pl.DeviceIdType.MESH,
                    ))
            for rdma in rdmas:
                rdma.start()
            for rdma in rdmas:
                rdma.wait()
            add_round(rs, first=False)

        finals = [
            pltpu.make_async_remote_copy(
                src_ref=abuf.at[R, 1], dst_ref=out_ref.at[pl.ds(0, half)],
                send_sem=fs.at[R], recv_sem=fr.at[R],
                device_id=(right,), device_id_type=pl.DeviceIdType.MESH,
            ),
            pltpu.make_async_remote_copy(
                src_ref=abuf.at[L, 1], dst_ref=out_ref.at[pl.ds(half, half)],
                send_sem=fs.at[L], recv_sem=fr.at[L],
                device_id=(left,), device_id_type=pl.DeviceIdType.MESH,
            ),
        ]
        for f in finals:
            f.start()
        for f in finals:
            f.wait()

    return pl.pallas_call(
        body,
        out_shape=jax.ShapeDtypeStruct((m, d_out), jnp.float32),
        in_specs=[
            pl.BlockSpec(memory_space=pl.ANY),
            pl.BlockSpec(memory_space=pl.ANY),
            pl.BlockSpec(memory_space=pl.ANY),
        ],
        out_specs=pl.BlockSpec(memory_space=pltpu.VMEM),
        scratch_shapes=[
            pltpu.VMEM((2, 2, half, d), jnp.float32),
            pltpu.VMEM((2, 2, half, n_exp), jnp.float32),
            pltpu.VMEM((2, 2, half, d_out), jnp.float32),
            pltpu.VMEM((2, d, d_out), jnp.float32),
            pltpu.SemaphoreType.DMA((4,)),
            pltpu.SemaphoreType.DMA((2,)),
            pltpu.SemaphoreType.DMA((2, 2)),
            pltpu.SemaphoreType.DMA((2, 2)),
            pltpu.SemaphoreType.DMA((2, 2)),
            pltpu.SemaphoreType.DMA((2, 2)),
            pltpu.SemaphoreType.DMA((2, 2)),
            pltpu.SemaphoreType.DMA((2, 2)),
            pltpu.SemaphoreType.DMA((2,)),
            pltpu.SemaphoreType.DMA((2,)),
        ],
        compiler_params=pltpu.CompilerParams(
            collective_id=0,
            vmem_limit_bytes=58 * 1024 * 1024,
        ),
    )(x, gw, expert_W)


# device time: 367027 ns/iter; 2.3507x vs baseline; 1.4467x over previous
import jax
import jax.numpy as jnp
from jax import lax
from jax.experimental import pallas as pl
from jax.experimental.pallas import tpu as pltpu

N_DEV = 4
E_LOCAL = 8
R, L = 0, 1


def kernel(x, router_W, route_idx, expert_W):
    m, d = x.shape
    n_exp = router_W.shape[1]
    d_out = expert_W.shape[2]
    half = m // 2

    scores = jnp.dot(x, router_W)
    probs = jax.nn.softmax(scores, axis=-1)
    top = jnp.take_along_axis(probs, route_idx, axis=1)
    gnorm = top / jnp.sum(top, axis=1, keepdims=True)
    onehot = route_idx[:, :, None] == jnp.arange(n_exp)[None, None, :]
    gw = jnp.sum(jnp.where(onehot, gnorm[:, :, None], 0.0), axis=1)
    gw = gw.astype(jnp.float32)

    def body(
        x_ref, gw_ref, ew_ref, out_ref,
        xbuf, gbuf, abuf, wbuf, ptmp,
        load_sem, w_sem,
        xs, xr, gs, gr, as_, ar, fs, fr,
    ):
        my = lax.axis_index("i")
        right = lax.rem(my + 1, N_DEV)
        left = lax.rem(my + N_DEV - 1, N_DEV)
        nbr = (right, left)
        base = my * E_LOCAL

        loads = [
            pltpu.make_async_copy(
                x_ref.at[pl.ds(0, half)], xbuf.at[R, 0], load_sem.at[0]),
            pltpu.make_async_copy(
                x_ref.at[pl.ds(half, half)], xbuf.at[L, 0], load_sem.at[1]),
            pltpu.make_async_copy(
                gw_ref.at[pl.ds(0, half)], gbuf.at[R, 0], load_sem.at[2]),
            pltpu.make_async_copy(
                gw_ref.at[pl.ds(half, half)], gbuf.at[L, 0], load_sem.at[3]),
        ]
        for cp in loads:
            cp.start()

        barrier = pltpu.get_barrier_semaphore()
        for tgt in (left, right):
            pl.semaphore_signal(
                barrier, inc=1,
                device_id=(tgt,), device_id_type=pl.DeviceIdType.MESH,
            )
        pl.semaphore_wait(barrier, 2)

        for cp in loads:
            cp.wait()

        col = lax.broadcasted_iota(jnp.int32, (half, n_exp), 1)

        def contrib(slot, into_ptmp):
            for dir_ in (R, L):
                z = jnp.zeros((half, d_out), jnp.float32)
                if into_ptmp:
                    ptmp[dir_] = z
                else:
                    abuf[dir_, slot] = z
            pltpu.make_async_copy(
                ew_ref.at[0], wbuf.at[0], w_sem.at[0]
            ).start()

            def step(e, carry):
                nxt = e + 1
                wslot = lax.rem(e, 2)
                nwslot = lax.rem(nxt, 2)

                @pl.when(nxt < E_LOCAL)
                def _():
                    pltpu.make_async_copy(
                        ew_ref.at[nxt], wbuf.at[nwslot], w_sem.at[nwslot]
                    ).start()

                pltpu.make_async_copy(
                    ew_ref.at[e], wbuf.at[wslot], w_sem.at[wslot]
                ).wait()
                for dir_ in (R, L):
                    w = jnp.sum(
                        jnp.where(col == base + e, gbuf[dir_, slot], 0.0),
                        axis=1, keepdims=True,
                    )
                    y = jnp.dot(
                        xbuf[dir_, slot], wbuf[wslot],
                        preferred_element_type=jnp.float32,
                    )
                    if into_ptmp:
                        ptmp[dir_] = ptmp[dir_] + w * y
                    else:
                        abuf[dir_, slot] = abuf[dir_, slot] + w * y
                return carry

            lax.fori_loop(0, E_LOCAL, step, 0)

        def make_xg(h):
            rs, ss = h % 2, (h + 1) % 2
            out = []
            for buf, send_sems, recv_sems in ((gbuf, gs, gr), (xbuf, xs, xr)):
                for dir_ in (R, L):
                    out.append(pltpu.make_async_remote_copy(
                        src_ref=buf.at[dir_, ss], dst_ref=buf.at[dir_, rs],
                        send_sem=send_sems.at[dir_, rs],
                        recv_sem=recv_sems.at[dir_, rs],
                        device_id=(nbr[dir_],),
                        device_id_type=pl.DeviceIdType.MESH,
                    ))
            return out

        def make_a(h):
            rs, ss = h % 2, (h + 1) % 2
            return [
                pltpu.make_async_remote_copy(
                    src_ref=abuf.at[dir_, ss], dst_ref=abuf.at[dir_, rs],
                    send_sem=as_.at[dir_, rs], recv_sem=ar.at[dir_, rs],
                    device_id=(nbr[dir_],),
                    device_id_type=pl.DeviceIdType.MESH,
                )
                for dir_ in (R, L)
            ]

        for rdma in make_xg(1):
            rdma.start()
        contrib(0, into_ptmp=False)

        for h in range(1, N_DEV):
            rs = h % 2
            for rdma in make_a(h):
                rdma.start()
            for rdma in make_xg(h):
                rdma.wait_recv()
                rdma.wait_send()
            if h < N_DEV - 1:
                for rdma in make_xg(h + 1):
                    rdma.start()
            contrib(rs, into_ptmp=True)
            for rdma in make_a(h):
                rdma.wait_recv()
                rdma.wait_send()
            for dir_ in (R, L):
                abuf[dir_, rs] = abuf[dir_, rs] + ptmp[dir_]

        finals = [
            pltpu.make_async_remote_copy(
                src_ref=abuf.at[R, 1], dst_ref=out_ref.at[pl.ds(0, half)],
                send_sem=fs.at[R], recv_sem=fr.at[R],
                device_id=(right,), device_id_type=pl.DeviceIdType.MESH,
            ),
            pltpu.make_async_remote_copy(
                src_ref=abuf.at[L, 1], dst_ref=out_ref.at[pl.ds(half, half)],
                send_sem=fs.at[L], recv_sem=fr.at[L],
                device_id=(left,), device_id_type=pl.DeviceIdType.MESH,
            ),
        ]
        for f in finals:
            f.start()
        for f in finals:
            f.wait()

    return pl.pallas_call(
        body,
        out_shape=jax.ShapeDtypeStruct((m, d_out), jnp.float32),
        in_specs=[
            pl.BlockSpec(memory_space=pl.ANY),
            pl.BlockSpec(memory_space=pl.ANY),
            pl.BlockSpec(memory_space=pl.ANY),
        ],
        out_specs=pl.BlockSpec(memory_space=pltpu.VMEM),
        scratch_shapes=[
            pltpu.VMEM((2, 2, half, d), jnp.float32),
            pltpu.VMEM((2, 2, half, n_exp), jnp.float32),
            pltpu.VMEM((2, 2, half, d_out), jnp.float32),
            pltpu.VMEM((2, d, d_out), jnp.float32),
            pltpu.VMEM((2, half, d_out), jnp.float32),
            pltpu.SemaphoreType.DMA((4,)),
            pltpu.SemaphoreType.DMA((2,)),
            pltpu.SemaphoreType.DMA((2, 2)),
            pltpu.SemaphoreType.DMA((2, 2)),
            pltpu.SemaphoreType.DMA((2, 2)),
            pltpu.SemaphoreType.DMA((2, 2)),
            pltpu.SemaphoreType.DMA((2, 2)),
            pltpu.SemaphoreType.DMA((2, 2)),
            pltpu.SemaphoreType.DMA((2,)),
            pltpu.SemaphoreType.DMA((2,)),
        ],
        compiler_params=pltpu.CompilerParams(
            collective_id=0,
            vmem_limit_bytes=61 * 1024 * 1024,
        ),
    )(x, gw, expert_W)
